# initial kernel scaffold (unmeasured)
import jax
import jax.numpy as jnp
from jax import lax
from jax.experimental import pallas as pl
from jax.experimental.pallas import tpu as pltpu


def kernel(
    x,
):
    def body(*refs):
        pass

    out_shape = jax.ShapeDtypeStruct(..., jnp.float32)
    return pl.pallas_call(body, out_shape=out_shape)(...)


# baseline (device time: 31562 ns/iter reference)
import jax
import jax.numpy as jnp
from jax import lax
from jax.experimental import pallas as pl
from jax.experimental.pallas import tpu as pltpu


def kernel(x):
    xb = x[0].astype(jnp.bfloat16)
    m, n2 = xb.shape
    nh = n2 // 2

    def body(x_ref, out_ref, comm_ref, send_sem, recv_sem):
        my_x = lax.axis_index("x")
        my_y = lax.axis_index("y")
        peer = (1 - my_x, my_y)

        barrier = pltpu.get_barrier_semaphore()
        pl.semaphore_signal(
            barrier, inc=1, device_id=peer, device_id_type=pl.DeviceIdType.MESH
        )
        pl.semaphore_wait(barrier, 1)

        comm_ref[0] = x_ref[:, pl.ds((1 - my_x) * nh, nh)]
        rdma = pltpu.make_async_remote_copy(
            src_ref=comm_ref.at[0],
            dst_ref=comm_ref.at[1],
            send_sem=send_sem,
            recv_sem=recv_sem,
            device_id=peer,
            device_id_type=pl.DeviceIdType.MESH,
        )
        rdma.start()
        rdma.wait()

        out_ref[...] = x_ref[:, pl.ds(my_x * nh, nh)] + comm_ref[1]

    return pl.pallas_call(
        body,
        out_shape=jax.ShapeDtypeStruct((m, nh), jnp.bfloat16),
        in_specs=[pl.BlockSpec(memory_space=pltpu.VMEM)],
        out_specs=pl.BlockSpec(memory_space=pltpu.VMEM),
        scratch_shapes=[
            pltpu.VMEM((2, m, nh), jnp.bfloat16),
            pltpu.SemaphoreType.DMA,
            pltpu.SemaphoreType.DMA,
        ],
        compiler_params=pltpu.CompilerParams(collective_id=0),
    )(xb)


# device time: 23725 ns/iter; 1.3303x vs baseline; 1.3303x over previous
import jax
import jax.numpy as jnp
from jax import lax
from jax.experimental import pallas as pl
from jax.experimental.pallas import tpu as pltpu

N_CHUNKS = 8


def kernel(x):
    xf = x[0]
    m, n2 = xf.shape
    nh = n2 // 2
    mh = m // 2
    rp = mh // N_CHUNKS

    def body(
        x_ref,
        out_ref,
        stage_ref,
        recv_ref,
        sendx_sems,
        recvx_sems,
        sendy_sems,
        recvy_sems,
    ):
        my_x = lax.axis_index("x")
        my_y = lax.axis_index("y")
        xpeer = (1 - my_x, my_y)
        ypeer = (my_x, 1 - my_y)

        barrier = pltpu.get_barrier_semaphore()
        for peer in (xpeer, ypeer):
            pl.semaphore_signal(
                barrier, inc=1, device_id=peer, device_id_type=pl.DeviceIdType.MESH
            )
        pl.semaphore_wait(barrier, 2)

        base = my_y * mh
        ccol = my_x * nh
        pcol = (1 - my_x) * nh

        x_rdmas = []
        for i in range(N_CHUNKS):
            stage_ref[i] = x_ref[
                pl.ds(base + i * rp, rp), pl.ds(pcol, nh)
            ].astype(jnp.bfloat16)
            rdma = pltpu.make_async_remote_copy(
                src_ref=stage_ref.at[i],
                dst_ref=recv_ref.at[i],
                send_sem=sendx_sems.at[i],
                recv_sem=recvx_sems.at[i],
                device_id=xpeer,
                device_id_type=pl.DeviceIdType.MESH,
            )
            rdma.start()
            x_rdmas.append(rdma)

        y_rdmas = []
        for i in range(N_CHUNKS):
            x_rdmas[i].wait_recv()
            rows = pl.ds(base + i * rp, rp)
            out_ref[rows, :] = (
                x_ref[rows, pl.ds(ccol, nh)].astype(jnp.bfloat16) + recv_ref[i]
            )
            rdma = pltpu.make_async_remote_copy(
                src_ref=out_ref.at[rows, :],
                dst_ref=out_ref.at[rows, :],
                send_sem=sendy_sems.at[i],
                recv_sem=recvy_sems.at[i],
                device_id=ypeer,
                device_id_type=pl.DeviceIdType.MESH,
            )
            rdma.start()
            y_rdmas.append(rdma)

        for i in range(N_CHUNKS):
            y_rdmas[i].wait_recv()
        for i in range(N_CHUNKS):
            x_rdmas[i].wait_send()
            y_rdmas[i].wait_send()

    return pl.pallas_call(
        body,
        out_shape=jax.ShapeDtypeStruct((m, nh), jnp.bfloat16),
        in_specs=[pl.BlockSpec(memory_space=pltpu.VMEM)],
        out_specs=pl.BlockSpec(memory_space=pltpu.VMEM),
        scratch_shapes=[
            pltpu.VMEM((N_CHUNKS, rp, nh), jnp.bfloat16),
            pltpu.VMEM((N_CHUNKS, rp, nh), jnp.bfloat16),
            pltpu.SemaphoreType.DMA((N_CHUNKS,)),
            pltpu.SemaphoreType.DMA((N_CHUNKS,)),
            pltpu.SemaphoreType.DMA((N_CHUNKS,)),
            pltpu.SemaphoreType.DMA((N_CHUNKS,)),
        ],
        compiler_params=pltpu.CompilerParams(collective_id=0),
    )(xf)


# device time: 21110 ns/iter; 1.4951x vs baseline; 1.1239x over previous
import jax
import jax.numpy as jnp
from jax import lax
from jax.experimental import pallas as pl
from jax.experimental.pallas import tpu as pltpu

N_CHUNKS = 8


def kernel(x):
    xf = x[0]
    m, n2 = xf.shape
    nh = n2 // 2
    mh = m // 2
    rp = mh // N_CHUNKS

    def body(
        x_ref,
        out_ref,
        stage_ref,
        recv_ref,
        sendx_sems,
        recvx_sems,
        sendy_sems,
        recvy_sems,
    ):
        my_x = lax.axis_index("x")
        my_y = lax.axis_index("y")
        xpeer = (1 - my_x, my_y)
        ypeer = (my_x, 1 - my_y)

        barrier = pltpu.get_barrier_semaphore()
        for peer in (xpeer, ypeer):
            pl.semaphore_signal(
                barrier, inc=1, device_id=peer, device_id_type=pl.DeviceIdType.MESH
            )
        pl.semaphore_wait(barrier, 2)

        base = my_y * mh
        ccol = my_x * nh
        pcol = (1 - my_x) * nh

        x_rdmas = []
        for i in range(N_CHUNKS):
            stage_ref[i] = x_ref[
                pl.ds(base + i * rp, rp), pl.ds(pcol, nh)
            ].astype(jnp.bfloat16)
            rdma = pltpu.make_async_remote_copy(
                src_ref=stage_ref.at[i],
                dst_ref=recv_ref.at[i],
                send_sem=sendx_sems.at[i],
                recv_sem=recvx_sems.at[i],
                device_id=xpeer,
                device_id_type=pl.DeviceIdType.MESH,
            )
            rdma.start()
            x_rdmas.append(rdma)

        y_rdmas = []
        for i in range(N_CHUNKS):
            x_rdmas[i].wait_recv()
            rows = pl.ds(base + i * rp, rp)
            out_ref[rows, :] = (
                x_ref[rows, pl.ds(ccol, nh)].astype(jnp.bfloat16) + recv_ref[i]
            )
        for i in range(N_CHUNKS):
            x_rdmas[i].wait_send()

    return pl.pallas_call(
        body,
        out_shape=jax.ShapeDtypeStruct((m, nh), jnp.bfloat16),
        in_specs=[pl.BlockSpec(memory_space=pltpu.VMEM)],
        out_specs=pl.BlockSpec(memory_space=pltpu.VMEM),
        scratch_shapes=[
            pltpu.VMEM((N_CHUNKS, rp, nh), jnp.bfloat16),
            pltpu.VMEM((N_CHUNKS, rp, nh), jnp.bfloat16),
            pltpu.SemaphoreType.DMA((N_CHUNKS,)),
            pltpu.SemaphoreType.DMA((N_CHUNKS,)),
            pltpu.SemaphoreType.DMA((N_CHUNKS,)),
            pltpu.SemaphoreType.DMA((N_CHUNKS,)),
        ],
        compiler_params=pltpu.CompilerParams(collective_id=0),
    )(xf)


# device time: 20681 ns/iter; 1.5261x vs baseline; 1.0207x over previous
import jax
import jax.numpy as jnp
from jax import lax
from jax.experimental import pallas as pl
from jax.experimental.pallas import tpu as pltpu

N_CHUNKS = 16
N_FETCH = 2


def kernel(x):
    xf = x[0]
    m, n2 = xf.shape
    nh = n2 // 2
    mh = m // 2
    rp = mh // N_CHUNKS
    fr = mh // N_FETCH

    my_y_outer = lax.axis_index("y")
    xh = lax.dynamic_slice_in_dim(
        xf.astype(jnp.bfloat16), my_y_outer * mh, mh, axis=0
    )

    def body(
        x_ref,
        out_ref,
        xloc_ref,
        red_ref,
        recv_ref,
        fetch_sems,
        store_sems,
        sendx_sems,
        recvx_sems,
        sendy_sems,
        recvy_sems,
    ):
        my_x = lax.axis_index("x")
        my_y = lax.axis_index("y")
        xpeer = (1 - my_x, my_y)
        ypeer = (my_x, 1 - my_y)

        base = my_y * mh
        ccol = my_x * nh
        pcol = (1 - my_x) * nh

        fetches = []
        for j in range(N_FETCH):
            cp = pltpu.make_async_copy(
                x_ref.at[pl.ds(j * fr, fr), pl.ds(ccol, nh)],
                xloc_ref.at[pl.ds(j * fr, fr), :],
                fetch_sems.at[j],
            )
            cp.start()
            fetches.append(cp)

        barrier = pltpu.get_barrier_semaphore()
        for peer in (xpeer, ypeer):
            pl.semaphore_signal(
                barrier, inc=1, device_id=peer, device_id_type=pl.DeviceIdType.MESH
            )
        pl.semaphore_wait(barrier, 2)

        x_rdmas = []
        for i in range(N_CHUNKS):
            rdma = pltpu.make_async_remote_copy(
                src_ref=x_ref.at[pl.ds(i * rp, rp), pl.ds(pcol, nh)],
                dst_ref=recv_ref.at[i],
                send_sem=sendx_sems.at[i],
                recv_sem=recvx_sems.at[i],
                device_id=xpeer,
                device_id_type=pl.DeviceIdType.MESH,
            )
            rdma.start()
            x_rdmas.append(rdma)

        y_rdmas = []
        stores = []
        chunks_per_fetch = N_CHUNKS // N_FETCH
        for i in range(N_CHUNKS):
            if i % chunks_per_fetch == 0:
                fetches[i // chunks_per_fetch].wait()
            x_rdmas[i].wait_recv()
            rows_loc = pl.ds(i * rp, rp)
            rows_out = pl.ds(base + i * rp, rp)
            red_ref[rows_loc, :] = xloc_ref[rows_loc, :] + recv_ref[i]
            rdma = pltpu.make_async_remote_copy(
                src_ref=red_ref.at[rows_loc, :],
                dst_ref=out_ref.at[rows_out, :],
                send_sem=sendy_sems.at[i],
                recv_sem=recvy_sems.at[i],
                device_id=ypeer,
                device_id_type=pl.DeviceIdType.MESH,
            )
            rdma.start()
            y_rdmas.append(rdma)
            if (i + 1) % chunks_per_fetch == 0:
                j = i // chunks_per_fetch
                st = pltpu.make_async_copy(
                    red_ref.at[pl.ds(j * fr, fr), :],
                    out_ref.at[pl.ds(base + j * fr, fr), :],
                    store_sems.at[j],
                )
                st.start()
                stores.append(st)

        for i in range(N_CHUNKS):
            y_rdmas[i].wait_recv()
        for j in range(N_FETCH):
            stores[j].wait()
        for i in range(N_CHUNKS):
            x_rdmas[i].wait_send()
            y_rdmas[i].wait_send()

    return pl.pallas_call(
        body,
        out_shape=jax.ShapeDtypeStruct((m, nh), jnp.bfloat16),
        in_specs=[pl.BlockSpec(memory_space=pl.ANY)],
        out_specs=pl.BlockSpec(memory_space=pl.ANY),
        scratch_shapes=[
            pltpu.VMEM((mh, nh), jnp.bfloat16),
            pltpu.VMEM((mh, nh), jnp.bfloat16),
            pltpu.VMEM((N_CHUNKS, rp, nh), jnp.bfloat16),
            pltpu.SemaphoreType.DMA((N_FETCH,)),
            pltpu.SemaphoreType.DMA((N_FETCH,)),
            pltpu.SemaphoreType.DMA((N_CHUNKS,)),
            pltpu.SemaphoreType.DMA((N_CHUNKS,)),
            pltpu.SemaphoreType.DMA((N_CHUNKS,)),
            pltpu.SemaphoreType.DMA((N_CHUNKS,)),
        ],
        compiler_params=pltpu.CompilerParams(collective_id=0),
    )(xh)
